# Initial kernel scaffold; baseline (speedup 1.0000x reference)
import functools

import jax
import jax.numpy as jnp
from jax import lax
from jax.experimental import pallas as pl
from jax.experimental.pallas import tpu as pltpu
from jax.experimental.pallas import tpu_sc as plsc

_N = 50000
_E = 1600000
_CUTOFF = 10.0
_KE = 14.399645351950548

_NC = 2
_NS = 16
_NW = _NC * _NS
_EPW = _E // _NW
_CH = 2000
_NCHUNK = _EPW // _CH
_NVEC = _CH // 16


def _rsqrt(a):
    i = lax.bitcast_convert_type(a, jnp.int32)
    i = jnp.int32(0x5F3759DF) - lax.shift_right_logical(i, 1)
    y = lax.bitcast_convert_type(i, jnp.float32)
    for _ in range(3):
        y = y * (1.5 - 0.5 * a * y * y)
    return y


def _chi_sc(d):
    x = d * (2.0 / _CUTOFF)
    x3 = x * x * x
    x4 = x3 * x
    x5 = x4 * x
    poly = 1.0 - 6.0 * x5 + 15.0 * x4 - 10.0 * x3
    p = jnp.where(d < (_CUTOFF * 0.5), poly, 0.0)
    inv_damped = _rsqrt(d * d + 1.0)
    inv_bare = _rsqrt(d * d)
    return p * inv_damped + (1.0 - p) * inv_bare


def _sc_body(qi_hbm, src_hbm, dst_hbm, dist_hbm, out_hbm,
             qi_v, acc_v, src_v, dst_v, dist_v):
    wid = lax.axis_index("c") * _NS + lax.axis_index("s")
    pltpu.sync_copy(qi_hbm, qi_v)

    def zero_body(j, carry):
        acc_v[pl.ds(j * 16, 16)] = jnp.zeros((16,), jnp.float32)
        return carry

    lax.fori_loop(0, _N // 16, zero_body, 0)

    base = wid * _EPW

    def chunk_body(g, carry):
        off = base + g * _CH
        pltpu.sync_copy(src_hbm.at[pl.ds(off, _CH)], src_v)
        pltpu.sync_copy(dst_hbm.at[pl.ds(off, _CH)], dst_v)
        pltpu.sync_copy(dist_hbm.at[pl.ds(off, _CH)], dist_v)

        def vec_body(j, c2):
            s = src_v[pl.ds(j * 16, 16)]
            t = dst_v[pl.ds(j * 16, 16)]
            d = dist_v[pl.ds(j * 16, 16)]
            q1 = plsc.load_gather(qi_v, [s])
            q2 = plsc.load_gather(qi_v, [t])
            pair = q1 * q2 * _chi_sc(d)
            plsc.addupdate_scatter(acc_v, [s], pair)
            return c2

        lax.fori_loop(0, _NVEC, vec_body, 0)
        return carry

    lax.fori_loop(0, _NCHUNK, chunk_body, 0)
    pltpu.sync_copy(acc_v, out_hbm.at[wid])


_sc_call = functools.partial(
    pl.kernel,
    mesh=plsc.VectorSubcoreMesh(core_axis_name="c", subcore_axis_name="s"),
    out_type=jax.ShapeDtypeStruct((_NW, _N), jnp.float32),
    scratch_types=[
        pltpu.VMEM((_N,), jnp.float32),
        pltpu.VMEM((_N,), jnp.float32),
        pltpu.VMEM((_CH,), jnp.int32),
        pltpu.VMEM((_CH,), jnp.int32),
        pltpu.VMEM((_CH,), jnp.float32),
    ],
)(_sc_body)


def _tc_reduce(x_ref, o_ref):
    o_ref[...] = jnp.sum(x_ref[...], axis=0) * (_KE * 0.5)


@jax.jit
def kernel(qi, edge_dist, edge_index):
    src = edge_index[0]
    dst = edge_index[1]
    part = _sc_call(qi, src, dst, edge_dist)
    return pl.pallas_call(
        _tc_reduce,
        out_shape=jax.ShapeDtypeStruct((_N,), jnp.float32),
    )(part)


# speedup vs baseline: 105.4456x; 105.4456x over previous
import functools

import jax
import jax.numpy as jnp
from jax import lax
from jax.experimental import pallas as pl
from jax.experimental.pallas import tpu as pltpu
from jax.experimental.pallas import tpu_sc as plsc

_N = 50000
_E = 1600000
_CUTOFF = 10.0
_KE = 14.399645351950548

_NC = 2
_NS = 16
_NW = _NC * _NS
_EPW = _E // _NW
_CH = 2000
_NCHUNK = _EPW // _CH
_NVEC = _CH // 16


def _rsqrt(a):
    i = lax.bitcast_convert_type(a, jnp.int32)
    i = jnp.int32(0x5F3759DF) - lax.shift_right_logical(i, 1)
    y = lax.bitcast_convert_type(i, jnp.float32)
    for _ in range(3):
        y = y * (1.5 - 0.5 * a * y * y)
    return y


def _chi_sc(d):
    x = d * (2.0 / _CUTOFF)
    x3 = x * x * x
    x4 = x3 * x
    x5 = x4 * x
    poly = 1.0 - 6.0 * x5 + 15.0 * x4 - 10.0 * x3
    p = jnp.where(d < (_CUTOFF * 0.5), poly, 0.0)
    inv_damped = _rsqrt(d * d + 1.0)
    inv_bare = _rsqrt(d * d)
    return p * inv_damped + (1.0 - p) * inv_bare


def _sc_body(qi_hbm, src_hbm, dst_hbm, dist_hbm, out_hbm,
             qi_v, acc_v, src_v, dst_v, dist_v):
    wid = lax.axis_index("c") * _NS + lax.axis_index("s")
    pltpu.sync_copy(qi_hbm, qi_v)

    def zero_body(j, carry):
        acc_v[pl.ds(j * 16, 16)] = jnp.zeros((16,), jnp.float32)
        return carry

    lax.fori_loop(0, _N // 16, zero_body, 0)

    base = wid * _EPW

    def chunk_body(g, carry):
        off = base + g * _CH
        pltpu.sync_copy(src_hbm.at[pl.ds(off, _CH)], src_v)
        pltpu.sync_copy(dst_hbm.at[pl.ds(off, _CH)], dst_v)
        pltpu.sync_copy(dist_hbm.at[pl.ds(off, _CH)], dist_v)

        def vec_body(j, c2):
            s = src_v[pl.ds(j * 16, 16)]
            t = dst_v[pl.ds(j * 16, 16)]
            d = dist_v[pl.ds(j * 16, 16)]
            q1 = plsc.load_gather(qi_v, [s])
            q2 = plsc.load_gather(qi_v, [t])
            pair = q1 * q2 * _chi_sc(d)
            plsc.addupdate_scatter(acc_v, [s], pair)
            return c2

        lax.fori_loop(0, _NVEC, vec_body, 0)
        return carry

    lax.fori_loop(0, _NCHUNK, chunk_body, 0)
    pltpu.sync_copy(acc_v, out_hbm.at[wid])


@functools.lru_cache(maxsize=1)
def _sc_call():
    return functools.partial(
        pl.kernel,
        mesh=plsc.VectorSubcoreMesh(core_axis_name="c", subcore_axis_name="s"),
        out_type=jax.ShapeDtypeStruct((_NW, _N), jnp.float32),
        compiler_params=pltpu.CompilerParams(needs_layout_passes=False),
        scratch_types=[
            pltpu.VMEM((_N,), jnp.float32),
            pltpu.VMEM((_N,), jnp.float32),
            pltpu.VMEM((_CH,), jnp.int32),
            pltpu.VMEM((_CH,), jnp.int32),
            pltpu.VMEM((_CH,), jnp.float32),
        ],
    )(_sc_body)


def _tc_reduce(x_ref, o_ref):
    o_ref[...] = jnp.sum(x_ref[...], axis=0) * (_KE * 0.5)


@jax.jit
def kernel(qi, edge_dist, edge_index):
    src = edge_index[0]
    dst = edge_index[1]
    part = _sc_call()(qi, src, dst, edge_dist)
    return pl.pallas_call(
        _tc_reduce,
        out_shape=jax.ShapeDtypeStruct((_N,), jnp.float32),
    )(part)


# speedup vs baseline: 137.3977x; 1.3030x over previous
import functools

import jax
import jax.numpy as jnp
from jax import lax
from jax.experimental import pallas as pl
from jax.experimental.pallas import tpu as pltpu
from jax.experimental.pallas import tpu_sc as plsc

_N = 50000
_E = 1600000
_CUTOFF = 10.0
_KE = 14.399645351950548

_NC = 2
_NS = 16
_NW = _NC * _NS
_EPW = _E // _NW
_CH = 2000
_NCHUNK = _EPW // _CH
_NVEC = _CH // 16


def _rsqrt(a):
    i = lax.bitcast_convert_type(a, jnp.int32)
    i = jnp.int32(0x5F3759DF) - lax.shift_right_logical(i, 1)
    y = lax.bitcast_convert_type(i, jnp.float32)
    for _ in range(3):
        y = y * (1.5 - 0.5 * a * y * y)
    return y


def _chi_sc(d):
    x = d * (2.0 / _CUTOFF)
    x3 = x * x * x
    x4 = x3 * x
    x5 = x4 * x
    poly = 1.0 - 6.0 * x5 + 15.0 * x4 - 10.0 * x3
    p = jnp.where(d < (_CUTOFF * 0.5), poly, 0.0)
    inv_damped = _rsqrt(d * d + 1.0)
    inv_bare = _rsqrt(d * d)
    return p * inv_damped + (1.0 - p) * inv_bare


def _sc_body(qi_hbm, src_hbm, dst_hbm, dist_hbm, out_hbm,
             qi_v, acc_v, src_v, dst_v, dist_v):
    wid = lax.axis_index("c") * _NS + lax.axis_index("s")
    pltpu.sync_copy(qi_hbm, qi_v)

    @plsc.parallel_loop(0, _N // 16, unroll=5)
    def _zero(j):
        acc_v[pl.ds(j * 16, 16)] = jnp.zeros((16,), jnp.float32)

    base = wid * _EPW

    def chunk_body(g, carry):
        off = base + g * _CH
        pltpu.sync_copy(src_hbm.at[pl.ds(off, _CH)], src_v)
        pltpu.sync_copy(dst_hbm.at[pl.ds(off, _CH)], dst_v)
        pltpu.sync_copy(dist_hbm.at[pl.ds(off, _CH)], dist_v)

        @plsc.parallel_loop(0, _NVEC, unroll=5)
        def _vec(j):
            s = src_v[pl.ds(j * 16, 16)]
            t = dst_v[pl.ds(j * 16, 16)]
            d = dist_v[pl.ds(j * 16, 16)]
            q1 = plsc.load_gather(qi_v, [s])
            q2 = plsc.load_gather(qi_v, [t])
            pair = q1 * q2 * _chi_sc(d)
            plsc.addupdate_scatter(acc_v, [s], pair)

        return carry

    lax.fori_loop(0, _NCHUNK, chunk_body, 0)
    pltpu.sync_copy(acc_v, out_hbm.at[wid])


@functools.lru_cache(maxsize=1)
def _sc_call():
    return functools.partial(
        pl.kernel,
        mesh=plsc.VectorSubcoreMesh(core_axis_name="c", subcore_axis_name="s"),
        out_type=jax.ShapeDtypeStruct((_NW, _N), jnp.float32),
        compiler_params=pltpu.CompilerParams(needs_layout_passes=False),
        scratch_types=[
            pltpu.VMEM((_N,), jnp.float32),
            pltpu.VMEM((_N,), jnp.float32),
            pltpu.VMEM((_CH,), jnp.int32),
            pltpu.VMEM((_CH,), jnp.int32),
            pltpu.VMEM((_CH,), jnp.float32),
        ],
    )(_sc_body)


def _tc_reduce(x_ref, o_ref):
    o_ref[...] = jnp.sum(x_ref[...], axis=0) * (_KE * 0.5)


@jax.jit
def kernel(qi, edge_dist, edge_index):
    src = edge_index[0]
    dst = edge_index[1]
    part = _sc_call()(qi, src, dst, edge_dist)
    return pl.pallas_call(
        _tc_reduce,
        out_shape=jax.ShapeDtypeStruct((_N,), jnp.float32),
    )(part)


# speedup vs baseline: 189.8154x; 1.3815x over previous
import functools

import jax
import jax.numpy as jnp
from jax import lax
from jax.experimental import pallas as pl
from jax.experimental.pallas import tpu as pltpu
from jax.experimental.pallas import tpu_sc as plsc

_N = 50000
_E = 1600000
_CUTOFF = 10.0
_KE = 14.399645351950548

_NC = 2
_NS = 16
_NW = _NC * _NS
_EPW = _E // _NW
_CH = 2000
_NCHUNK = _EPW // _CH
_NVEC = _CH // 16


def _rsqrt(a):
    i = lax.bitcast_convert_type(a, jnp.int32)
    i = jnp.int32(0x5F3759DF) - lax.shift_right_logical(i, 1)
    y = lax.bitcast_convert_type(i, jnp.float32)
    for _ in range(3):
        y = y * (1.5 - 0.5 * a * y * y)
    return y


def _chi_sc(d):
    x = d * (2.0 / _CUTOFF)
    x3 = x * x * x
    x4 = x3 * x
    x5 = x4 * x
    poly = 1.0 - 6.0 * x5 + 15.0 * x4 - 10.0 * x3
    p = jnp.where(d < (_CUTOFF * 0.5), poly, 0.0)
    inv_damped = _rsqrt(d * d + 1.0)
    inv_bare = _rsqrt(d * d)
    return p * inv_damped + (1.0 - p) * inv_bare


def _sc_body(qi_hbm, src_hbm, dst_hbm, dist_hbm, out_hbm,
             qi_v, acc_v,
             src_a, dst_a, dist_a, src_b, dst_b, dist_b,
             semq, sema, semb):
    wid = lax.axis_index("c") * _NS + lax.axis_index("s")
    base = wid * _EPW
    buf_a = (src_a, dst_a, dist_a)
    buf_b = (src_b, dst_b, dist_b)

    def issue(c, bufs, sem):
        off = base + c * _CH
        pltpu.async_copy(src_hbm.at[pl.ds(off, _CH)], bufs[0], sem)
        pltpu.async_copy(dst_hbm.at[pl.ds(off, _CH)], bufs[1], sem)
        pltpu.async_copy(dist_hbm.at[pl.ds(off, _CH)], bufs[2], sem)

    def drain(c, bufs, sem):
        off = base + c * _CH
        pltpu.make_async_copy(src_hbm.at[pl.ds(off, _CH)], bufs[0], sem).wait()
        pltpu.make_async_copy(dst_hbm.at[pl.ds(off, _CH)], bufs[1], sem).wait()
        pltpu.make_async_copy(dist_hbm.at[pl.ds(off, _CH)], bufs[2], sem).wait()

    def compute(bufs):
        @plsc.parallel_loop(0, _NVEC, unroll=5)
        def _vec(j):
            s = bufs[0][pl.ds(j * 16, 16)]
            t = bufs[1][pl.ds(j * 16, 16)]
            d = bufs[2][pl.ds(j * 16, 16)]
            q1 = plsc.load_gather(qi_v, [s])
            q2 = plsc.load_gather(qi_v, [t])
            pair = q1 * q2 * _chi_sc(d)
            plsc.addupdate_scatter(acc_v, [s], pair)

    qi_cp = pltpu.async_copy(qi_hbm, qi_v, semq)
    issue(0, buf_a, sema)

    @plsc.parallel_loop(0, _N // 16, unroll=5)
    def _zero(j):
        acc_v[pl.ds(j * 16, 16)] = jnp.zeros((16,), jnp.float32)

    qi_cp.wait()

    def chunk_pair(k, carry):
        c = 2 * k
        drain(c, buf_a, sema)
        issue(c + 1, buf_b, semb)
        compute(buf_a)
        drain(c + 1, buf_b, semb)
        issue(c + 2, buf_a, sema)
        compute(buf_b)
        return carry

    lax.fori_loop(0, (_NCHUNK - 1) // 2, chunk_pair, 0)
    drain(_NCHUNK - 1, buf_a, sema)
    compute(buf_a)
    pltpu.sync_copy(acc_v, out_hbm.at[wid])


@functools.lru_cache(maxsize=1)
def _sc_call():
    return functools.partial(
        pl.kernel,
        mesh=plsc.VectorSubcoreMesh(core_axis_name="c", subcore_axis_name="s"),
        out_type=jax.ShapeDtypeStruct((_NW, _N), jnp.float32),
        compiler_params=pltpu.CompilerParams(needs_layout_passes=False),
        scratch_types=[
            pltpu.VMEM((_N,), jnp.float32),
            pltpu.VMEM((_N,), jnp.float32),
            pltpu.VMEM((_CH,), jnp.int32),
            pltpu.VMEM((_CH,), jnp.int32),
            pltpu.VMEM((_CH,), jnp.float32),
            pltpu.VMEM((_CH,), jnp.int32),
            pltpu.VMEM((_CH,), jnp.int32),
            pltpu.VMEM((_CH,), jnp.float32),
            pltpu.SemaphoreType.DMA,
            pltpu.SemaphoreType.DMA,
            pltpu.SemaphoreType.DMA,
        ],
    )(_sc_body)


def _tc_reduce(x_ref, o_ref):
    o_ref[...] = jnp.sum(x_ref[...], axis=0) * (_KE * 0.5)


@jax.jit
def kernel(qi, edge_dist, edge_index):
    src = edge_index[0]
    dst = edge_index[1]
    part = _sc_call()(qi, src, dst, edge_dist)
    return pl.pallas_call(
        _tc_reduce,
        out_shape=jax.ShapeDtypeStruct((_N,), jnp.float32),
    )(part)


# speedup vs baseline: 270.2897x; 1.4240x over previous
import functools

import jax
import jax.numpy as jnp
from jax import lax
from jax.experimental import pallas as pl
from jax.experimental.pallas import tpu as pltpu
from jax.experimental.pallas import tpu_sc as plsc

_N = 50000
_E = 1600000
_CUTOFF = 10.0
_KE = 14.399645351950548

_NC = 2
_NS = 16
_NW = _NC * _NS
_EPW = _E // _NW
_CH = 2000
_NCHUNK = _EPW // _CH
_NVEC = _CH // 16

_WBLK = 160000


def _chi_tc(d):
    x = d * (2.0 / _CUTOFF)
    x3 = x * x * x
    x4 = x3 * x
    x5 = x4 * x
    poly = 1.0 - 6.0 * x5 + 15.0 * x4 - 10.0 * x3
    p = jnp.where(d < (_CUTOFF * 0.5), poly, 0.0)
    return p / jnp.sqrt(d * d + 1.0) + (1.0 - p) / d


def _chi_body(d_ref, w_ref):
    w_ref[...] = _chi_tc(d_ref[...])


def _sc_body(qi_hbm, edge_hbm, w_hbm, out_hbm,
             qi_v, acc_v,
             src_a, dst_a, w_a, src_b, dst_b, w_b,
             semq, sema, semb):
    wid = lax.axis_index("c") * _NS + lax.axis_index("s")
    base = wid * _EPW
    buf_a = (src_a, dst_a, w_a)
    buf_b = (src_b, dst_b, w_b)

    def issue(c, bufs, sem):
        off = base + c * _CH
        pltpu.async_copy(edge_hbm.at[pl.ds(off, _CH)], bufs[0], sem)
        pltpu.async_copy(edge_hbm.at[pl.ds(_E + off, _CH)], bufs[1], sem)
        pltpu.async_copy(w_hbm.at[pl.ds(off, _CH)], bufs[2], sem)

    def drain(c, bufs, sem):
        off = base + c * _CH
        pltpu.make_async_copy(edge_hbm.at[pl.ds(off, _CH)], bufs[0], sem).wait()
        pltpu.make_async_copy(edge_hbm.at[pl.ds(_E + off, _CH)], bufs[1], sem).wait()
        pltpu.make_async_copy(w_hbm.at[pl.ds(off, _CH)], bufs[2], sem).wait()

    def compute(bufs):
        @plsc.parallel_loop(0, _NVEC, unroll=5)
        def _vec(j):
            s = bufs[0][pl.ds(j * 16, 16)]
            t = bufs[1][pl.ds(j * 16, 16)]
            w = bufs[2][pl.ds(j * 16, 16)]
            q1 = plsc.load_gather(qi_v, [s])
            q2 = plsc.load_gather(qi_v, [t])
            plsc.addupdate_scatter(acc_v, [s], q1 * q2 * w)

    qi_cp = pltpu.async_copy(qi_hbm, qi_v, semq)
    issue(0, buf_a, sema)

    @plsc.parallel_loop(0, _N // 16, unroll=5)
    def _zero(j):
        acc_v[pl.ds(j * 16, 16)] = jnp.zeros((16,), jnp.float32)

    qi_cp.wait()

    def chunk_pair(k, carry):
        c = 2 * k
        drain(c, buf_a, sema)
        issue(c + 1, buf_b, semb)
        compute(buf_a)
        drain(c + 1, buf_b, semb)
        issue(c + 2, buf_a, sema)
        compute(buf_b)
        return carry

    lax.fori_loop(0, (_NCHUNK - 1) // 2, chunk_pair, 0)
    drain(_NCHUNK - 1, buf_a, sema)
    compute(buf_a)
    pltpu.sync_copy(acc_v, out_hbm.at[wid])


@functools.lru_cache(maxsize=1)
def _sc_call():
    return functools.partial(
        pl.kernel,
        mesh=plsc.VectorSubcoreMesh(core_axis_name="c", subcore_axis_name="s"),
        out_type=jax.ShapeDtypeStruct((_NW, _N), jnp.float32),
        compiler_params=pltpu.CompilerParams(needs_layout_passes=False),
        scratch_types=[
            pltpu.VMEM((_N,), jnp.float32),
            pltpu.VMEM((_N,), jnp.float32),
            pltpu.VMEM((_CH,), jnp.int32),
            pltpu.VMEM((_CH,), jnp.int32),
            pltpu.VMEM((_CH,), jnp.float32),
            pltpu.VMEM((_CH,), jnp.int32),
            pltpu.VMEM((_CH,), jnp.int32),
            pltpu.VMEM((_CH,), jnp.float32),
            pltpu.SemaphoreType.DMA,
            pltpu.SemaphoreType.DMA,
            pltpu.SemaphoreType.DMA,
        ],
    )(_sc_body)


def _tc_reduce(x_ref, o_ref):
    o_ref[...] = jnp.sum(x_ref[...], axis=0) * (_KE * 0.5)


@jax.jit
def kernel(qi, edge_dist, edge_index):
    w = pl.pallas_call(
        _chi_body,
        out_shape=jax.ShapeDtypeStruct((_E,), jnp.float32),
    )(edge_dist)
    part = _sc_call()(qi, edge_index.reshape(-1), w)
    return pl.pallas_call(
        _tc_reduce,
        out_shape=jax.ShapeDtypeStruct((_N,), jnp.float32),
    )(part)


# speedup vs baseline: 365.2557x; 1.3513x over previous
import functools

import jax
import jax.numpy as jnp
from jax import lax
from jax.experimental import pallas as pl
from jax.experimental.pallas import tpu as pltpu
from jax.experimental.pallas import tpu_sc as plsc

_N = 50000
_E = 1600000
_CUTOFF = 10.0
_KE = 14.399645351950548

_NC = 2
_NS = 16
_NW = _NC * _NS
_EPW = _E // _NW
_CH = 2000
_NCHUNK = _EPW // _CH
_NVEC = _CH // 16

_WBLK = 160000


def _chi_tc(d):
    x = d * (2.0 / _CUTOFF)
    x3 = x * x * x
    x4 = x3 * x
    x5 = x4 * x
    poly = 1.0 - 6.0 * x5 + 15.0 * x4 - 10.0 * x3
    p = jnp.where(d < (_CUTOFF * 0.5), poly, 0.0)
    return p / jnp.sqrt(d * d + 1.0) + (1.0 - p) / d


def _pre_body(idx_ref, d_ref, src_ref, dst_ref, w_ref):
    src_ref[...] = idx_ref[0, :]
    dst_ref[...] = idx_ref[1, :]
    w_ref[...] = _chi_tc(d_ref[...])


def _sc_body(qi_hbm, src_hbm, dst_hbm, w_hbm, out_hbm,
             qi_v, acc_v,
             src_a, dst_a, w_a, src_b, dst_b, w_b,
             semq, sema, semb):
    wid = lax.axis_index("c") * _NS + lax.axis_index("s")
    base = wid * _EPW
    buf_a = (src_a, dst_a, w_a)
    buf_b = (src_b, dst_b, w_b)

    def issue(c, bufs, sem):
        off = base + c * _CH
        pltpu.async_copy(src_hbm.at[pl.ds(off, _CH)], bufs[0], sem)
        pltpu.async_copy(dst_hbm.at[pl.ds(off, _CH)], bufs[1], sem)
        pltpu.async_copy(w_hbm.at[pl.ds(off, _CH)], bufs[2], sem)

    def drain(c, bufs, sem):
        off = base + c * _CH
        pltpu.make_async_copy(src_hbm.at[pl.ds(off, _CH)], bufs[0], sem).wait()
        pltpu.make_async_copy(dst_hbm.at[pl.ds(off, _CH)], bufs[1], sem).wait()
        pltpu.make_async_copy(w_hbm.at[pl.ds(off, _CH)], bufs[2], sem).wait()

    def compute(bufs):
        @plsc.parallel_loop(0, _NVEC, unroll=5)
        def _vec(j):
            s = bufs[0][pl.ds(j * 16, 16)]
            t = bufs[1][pl.ds(j * 16, 16)]
            w = bufs[2][pl.ds(j * 16, 16)]
            q1 = plsc.load_gather(qi_v, [s])
            q2 = plsc.load_gather(qi_v, [t])
            plsc.addupdate_scatter(acc_v, [s], q1 * q2 * w)

    qi_cp = pltpu.async_copy(qi_hbm, qi_v, semq)
    issue(0, buf_a, sema)

    @plsc.parallel_loop(0, _N // 16, unroll=5)
    def _zero(j):
        acc_v[pl.ds(j * 16, 16)] = jnp.zeros((16,), jnp.float32)

    qi_cp.wait()

    def chunk_pair(k, carry):
        c = 2 * k
        drain(c, buf_a, sema)
        issue(c + 1, buf_b, semb)
        compute(buf_a)
        drain(c + 1, buf_b, semb)
        issue(c + 2, buf_a, sema)
        compute(buf_b)
        return carry

    lax.fori_loop(0, (_NCHUNK - 1) // 2, chunk_pair, 0)
    drain(_NCHUNK - 1, buf_a, sema)
    compute(buf_a)
    pltpu.sync_copy(acc_v, out_hbm.at[wid])


@functools.lru_cache(maxsize=1)
def _sc_call():
    return functools.partial(
        pl.kernel,
        mesh=plsc.VectorSubcoreMesh(core_axis_name="c", subcore_axis_name="s"),
        out_type=jax.ShapeDtypeStruct((_NW, _N), jnp.float32),
        compiler_params=pltpu.CompilerParams(needs_layout_passes=False),
        scratch_types=[
            pltpu.VMEM((_N,), jnp.float32),
            pltpu.VMEM((_N,), jnp.float32),
            pltpu.VMEM((_CH,), jnp.int32),
            pltpu.VMEM((_CH,), jnp.int32),
            pltpu.VMEM((_CH,), jnp.float32),
            pltpu.VMEM((_CH,), jnp.int32),
            pltpu.VMEM((_CH,), jnp.int32),
            pltpu.VMEM((_CH,), jnp.float32),
            pltpu.SemaphoreType.DMA,
            pltpu.SemaphoreType.DMA,
            pltpu.SemaphoreType.DMA,
        ],
    )(_sc_body)


def _tc_reduce(x_ref, o_ref):
    o_ref[...] = jnp.sum(x_ref[...], axis=0) * (_KE * 0.5)


@jax.jit
def kernel(qi, edge_dist, edge_index):
    src, dst, w = pl.pallas_call(
        _pre_body,
        out_shape=(
            jax.ShapeDtypeStruct((_E,), jnp.int32),
            jax.ShapeDtypeStruct((_E,), jnp.int32),
            jax.ShapeDtypeStruct((_E,), jnp.float32),
        ),
    )(edge_index, edge_dist)
    part = _sc_call()(qi, src, dst, w)
    return pl.pallas_call(
        _tc_reduce,
        out_shape=jax.ShapeDtypeStruct((_N,), jnp.float32),
    )(part)
